# Initial kernel scaffold; baseline (speedup 1.0000x reference)
import jax
import jax.numpy as jnp
from jax.experimental import pallas as pl


def kernel(h, edge_index, delete_eids, W1, a1, W2, a2, Wp, bp):
    raise NotImplementedError("write your pallas kernel here")


# speedup vs baseline: 13.5639x; 13.5639x over previous
import dataclasses
import functools

import jax
import jax.numpy as jnp
from jax import lax
from jax.experimental import pallas as pl
from jax.experimental.pallas import tpu as pltpu
from jax.experimental.pallas import tpu_sc as plsc

N = 10000
E = 320000
IN = 128
HID = 64
HEADS = 2
OUT = 64
C = 2
DEL = 1000

L = 16
NTILES = 32
EPT = E // NTILES
CHUNK = 80
NCHUNK = EPT // CHUNK
DELP = 1008

W1EXT = 160
W2EXT = 80

_f32 = jnp.float32
_i32 = jnp.int32

_SC_PARAMS = pltpu.CompilerParams()
for _field, _val in (("needs_layout_passes", False),
                     ("use_tc_tiling_on_sc", False)):
    if _field in pltpu.CompilerParams.__dataclass_fields__:
        _SC_PARAMS = dataclasses.replace(_SC_PARAMS, **{_field: _val})


def _t0_body(h_ref, w1_ref, a1_ref, z1ext_ref, sd_ref):
    z1 = jnp.dot(h_ref[...], w1_ref[...], preferred_element_type=_f32)
    s1 = jnp.dot(z1, a1_ref[...], preferred_element_type=_f32)
    z1ext_ref[:, :IN] = z1
    ones = jnp.ones((N, 1), _f32)
    zp = jnp.zeros((N, 13), _f32)
    z1ext_ref[:, IN:IN + L] = jnp.concatenate(
        [ones, s1[:, 0:1], s1[:, 1:2], zp], axis=1)
    z1ext_ref[:, IN + L:] = jnp.concatenate(
        [ones, zp, jnp.zeros((N, 2), _f32)], axis=1)
    sd_ref[...] = jnp.concatenate(
        [s1[:, 2:4], jnp.zeros((N, 14), _f32)], axis=1)


def _t1_body(accp_ref, w2_ref, a2_ref, z2ext_ref, sd_ref):
    acc = accp_ref[0] + accp_ref[1]
    d0 = jnp.maximum(acc[:, IN:IN + 1], 1e-16)
    d1 = jnp.maximum(acc[:, IN + L:IN + L + 1], 1e-16)
    h1a = acc[:, :HID] / d0
    h1b = acc[:, HID:IN] / d1
    h1 = jnp.concatenate([h1a, h1b], axis=1)
    h1 = jnp.where(h1 >= 0, h1, 0.01 * h1)
    z2 = jnp.dot(h1, w2_ref[...], preferred_element_type=_f32)
    s2 = jnp.dot(z2, a2_ref[...], preferred_element_type=_f32)
    z2ext_ref[:, :OUT] = z2
    ones = jnp.ones((N, 1), _f32)
    z2ext_ref[:, OUT:] = jnp.concatenate(
        [ones, s2[:, 0:1], jnp.zeros((N, 14), _f32)], axis=1)
    sd_ref[...] = jnp.concatenate(
        [s2[:, 1:2], jnp.zeros((N, 15), _f32)], axis=1)


def _t2_body(accp_ref, wpt_ref, wpb_ref, bp_ref, p_ref):
    acc = accp_ref[0] + accp_ref[1]
    den = jnp.maximum(acc[:, OUT:OUT + 1], 1e-16)
    h2 = acc[:, :OUT] / den
    ps = jnp.dot(h2, wpt_ref[...], preferred_element_type=_f32)
    pd = jnp.dot(h2, wpb_ref[...], preferred_element_type=_f32) + bp_ref[...]
    p_ref[...] = jnp.concatenate([ps, pd], axis=1)


def _splat_i32(x):
    return jnp.broadcast_to(jnp.asarray(x, _i32), (L,))


def _leaky(x):
    return jnp.where(x >= 0, x, 0.01 * x)


def _build_keep(keep_v, del_v, elo):
    ones = jnp.full((L,), 1.0, _f32)
    zeros = jnp.full((L,), 0.0, _f32)

    @pl.loop(0, EPT, step=L)
    def _(i):
        keep_v[pl.ds(i, L)] = ones

    @pl.loop(0, DELP, step=L)
    def _(i):
        d16 = del_v[pl.ds(i, L)]
        loc = d16 - elo
        m = (loc >= 0) & (loc < EPT)
        locc = jnp.clip(loc, 0, EPT - 1)
        plsc.store_scatter(keep_v, [locc], zeros, mask=m)


def _bcast_f32(ref, e):
    return plsc.load_gather(ref, [jnp.broadcast_to(e, (L,)).astype(_i32)])


def _make_edge_pass(width, heads):
    nslice = width // L
    aux = width - 2 * L if heads == 2 else width - L
    mesh = plsc.VectorSubcoreMesh(core_axis_name="c", subcore_axis_name="s")

    @functools.partial(
        pl.kernel,
        out_type=jax.ShapeDtypeStruct((2, N, width), _f32),
        mesh=mesh,
        scratch_types=[
            pltpu.VMEM((EPT,), _f32),
            pltpu.VMEM((DELP,), _i32),
            pltpu.VMEM((CHUNK,), _i32),
            pltpu.VMEM((CHUNK,), _i32),
            pltpu.VMEM((CHUNK,), _f32),
            pltpu.VMEM((CHUNK,), _f32),
            pltpu.VMEM((CHUNK, width), _f32),
            pltpu.VMEM((CHUNK, L), _f32),
            pltpu.VMEM_SHARED((N, width), _f32),
        ],
        compiler_params=_SC_PARAMS,
    )
    def edge_pass(zext_hbm, sd_hbm, src_hbm, dst_hbm, del_hbm, accp_hbm,
                  keep_v, del_v, src_v, dst_v, e0_v, e1_v, rows_v,
                  sdrows_v, acc_sh):
        cid = lax.axis_index("c")
        sid = lax.axis_index("s")
        wid = sid * 2 + cid
        elo = wid * EPT

        pltpu.sync_copy(del_hbm, del_v)
        _build_keep(keep_v, del_v, elo)

        zeros = jnp.full((L,), 0.0, _f32)

        @pl.loop(0, CHUNK)
        def _(i):
            for s in range(nslice):
                rows_v[i, pl.ds(s * L, L)] = zeros

        @pl.when(sid < 10)
        def _():
            for k in range(12):
                pltpu.sync_copy(
                    rows_v, acc_sh.at[pl.ds(sid * 1000 + k * CHUNK, CHUNK)])
            pltpu.sync_copy(rows_v.at[pl.ds(0, 40)],
                            acc_sh.at[pl.ds(sid * 1000 + 960, 40)])
        plsc.subcore_barrier()

        cs0 = _splat_i32(aux + 1)
        cs1 = _splat_i32(aux + 2)
        cd0 = _splat_i32(0)
        cd1 = _splat_i32(1)
        iota = lax.iota(_i32, L)

        @pl.loop(0, NCHUNK)
        def chunk(c):
            pltpu.sync_copy(src_hbm.at[wid].at[c], src_v)
            pltpu.sync_copy(dst_hbm.at[wid].at[c], dst_v)
            pltpu.sync_copy(zext_hbm.at[src_v], rows_v)
            pltpu.sync_copy(sd_hbm.at[dst_v], sdrows_v)

            @pl.loop(0, CHUNK // L)
            def grp(g):
                idx16 = iota + g * L
                k16 = keep_v[pl.ds(c * CHUNK + g * L, L)]
                e0 = (plsc.load_gather(rows_v, [idx16, cs0])
                      + plsc.load_gather(sdrows_v, [idx16, cd0]))
                e0_v[pl.ds(g * L, L)] = jnp.exp(_leaky(e0)) * k16
                if heads == 2:
                    e1 = (plsc.load_gather(rows_v, [idx16, cs1])
                          + plsc.load_gather(sdrows_v, [idx16, cd1]))
                    e1_v[pl.ds(g * L, L)] = jnp.exp(_leaky(e1)) * k16

            @pl.loop(0, CHUNK)
            def srow(e):
                b0 = _bcast_f32(e0_v, e)
                if heads == 2:
                    b1 = _bcast_f32(e1_v, e)
                    for s in range(nslice):
                        b = b0 if (s < 4 or s == nslice - 2) else b1
                        rows_v[e, pl.ds(s * L, L)] = rows_v[e, pl.ds(s * L, L)] * b
                else:
                    for s in range(nslice):
                        rows_v[e, pl.ds(s * L, L)] = rows_v[e, pl.ds(s * L, L)] * b0

            pltpu.sync_copy(rows_v, acc_sh.at[dst_v], add=True)

        plsc.subcore_barrier()

        @pl.when(sid < 10)
        def _():
            for k in range(12):
                r0 = sid * 1000 + k * CHUNK
                pltpu.sync_copy(acc_sh.at[pl.ds(r0, CHUNK)], rows_v)
                pltpu.sync_copy(rows_v, accp_hbm.at[cid].at[pl.ds(r0, CHUNK)])
            r0 = sid * 1000 + 960
            pltpu.sync_copy(acc_sh.at[pl.ds(r0, 40)], rows_v.at[pl.ds(0, 40)])
            pltpu.sync_copy(rows_v.at[pl.ds(0, 40)],
                            accp_hbm.at[cid].at[pl.ds(r0, 40)])

    return edge_pass


_edge_pass1 = _make_edge_pass(W1EXT, 2)
_edge_pass2 = _make_edge_pass(W2EXT, 1)


def _make_score_pass():
    mesh = plsc.VectorSubcoreMesh(core_axis_name="c", subcore_axis_name="s")

    @functools.partial(
        pl.kernel,
        out_type=jax.ShapeDtypeStruct((E, 2), _f32),
        mesh=mesh,
        scratch_types=[
            pltpu.VMEM((N, 4), _f32),
            pltpu.VMEM((CHUNK,), _i32),
            pltpu.VMEM((CHUNK,), _i32),
            pltpu.VMEM((CHUNK, 2), _f32),
        ],
        compiler_params=_SC_PARAMS,
    )
    def score_pass(p_hbm, src_hbm, dst_hbm, score_hbm, p_v, src_v, dst_v,
                   out_buf):
        cid = lax.axis_index("c")
        sid = lax.axis_index("s")
        wid = sid * 2 + cid
        elo = wid * EPT

        pltpu.sync_copy(p_hbm, p_v)

        c0 = _splat_i32(0)
        c1 = _splat_i32(1)
        c2 = _splat_i32(2)
        c3 = _splat_i32(3)
        iota = lax.iota(_i32, L)

        @pl.loop(0, NCHUNK)
        def chunk(c):
            pltpu.sync_copy(src_hbm.at[wid].at[c], src_v)
            pltpu.sync_copy(dst_hbm.at[wid].at[c], dst_v)

            @pl.loop(0, CHUNK // L)
            def grp(g):
                s16 = src_v[pl.ds(g * L, L)]
                d16 = dst_v[pl.ds(g * L, L)]
                v0 = (plsc.load_gather(p_v, [s16, c0])
                      + plsc.load_gather(p_v, [d16, c2]))
                v1 = (plsc.load_gather(p_v, [s16, c1])
                      + plsc.load_gather(p_v, [d16, c3]))
                rows = iota + g * L
                plsc.store_scatter(out_buf, [rows, c0], v0)
                plsc.store_scatter(out_buf, [rows, c1], v1)

            pltpu.sync_copy(out_buf, score_hbm.at[pl.ds(elo + c * CHUNK, CHUNK)])

    return score_pass


_score_pass = _make_score_pass()


def _t0(h, w1cat, a1m):
    return pl.pallas_call(
        _t0_body,
        out_shape=[
            jax.ShapeDtypeStruct((N, W1EXT), _f32),
            jax.ShapeDtypeStruct((N, L), _f32),
        ],
    )(h, w1cat, a1m)


def _t1(accp, w2, a2m):
    return pl.pallas_call(
        _t1_body,
        out_shape=[
            jax.ShapeDtypeStruct((N, W2EXT), _f32),
            jax.ShapeDtypeStruct((N, L), _f32),
        ],
    )(accp, w2, a2m)


def _t2(accp, wpt, wpb, bp2):
    return pl.pallas_call(
        _t2_body,
        out_shape=jax.ShapeDtypeStruct((N, 4), _f32),
    )(accp, wpt, wpb, bp2)


@jax.jit
def kernel(h, edge_index, delete_eids, W1, a1, W2, a2, Wp, bp):
    src_r = edge_index[0].astype(_i32).reshape(NTILES, NCHUNK, CHUNK)
    dst_r = edge_index[1].astype(_i32).reshape(NTILES, NCHUNK, CHUNK)
    del_pad = jnp.concatenate(
        [delete_eids.astype(_i32),
         jnp.broadcast_to(delete_eids[:1].astype(_i32), (DELP - DEL,))])

    w1cat = jnp.concatenate([W1[0], W1[1]], axis=1)
    a1m = jnp.zeros((IN, 4), _f32)
    a1m = a1m.at[:HID, 0].set(a1[0, :HID, 0])
    a1m = a1m.at[HID:, 1].set(a1[1, :HID, 0])
    a1m = a1m.at[:HID, 2].set(a1[0, HID:, 0])
    a1m = a1m.at[HID:, 3].set(a1[1, HID:, 0])
    a2m = jnp.stack([a2[:OUT, 0], a2[OUT:, 0]], axis=1)
    wpt = Wp[:OUT]
    wpb = Wp[OUT:]
    bp2 = bp.reshape(1, C)

    z1ext, s1 = _t0(h, w1cat, a1m)
    accp1 = _edge_pass1(z1ext, s1, src_r, dst_r, del_pad)
    z2ext, s2 = _t1(accp1, W2, a2m)
    accp2 = _edge_pass2(z2ext, s2, src_r, dst_r, del_pad)
    p = _t2(accp2, wpt, wpb, bp2)
    score = _score_pass(p, src_r, dst_r)
    return score


# speedup vs baseline: 18.5341x; 1.3664x over previous
import dataclasses
import functools

import jax
import jax.numpy as jnp
from jax import lax
from jax.experimental import pallas as pl
from jax.experimental.pallas import tpu as pltpu
from jax.experimental.pallas import tpu_sc as plsc

N = 10000
E = 320000
IN = 128
HID = 64
HEADS = 2
OUT = 64
C = 2
DEL = 1000

L = 16
NTILES = 32
EPT = E // NTILES
CHUNK = 80
NCHUNK = EPT // CHUNK
DELP = 1008

W1EXT = 160
W2EXT = 80

_f32 = jnp.float32
_i32 = jnp.int32

_SC_PARAMS = pltpu.CompilerParams()
for _field, _val in (("needs_layout_passes", False),
                     ("use_tc_tiling_on_sc", False)):
    if _field in pltpu.CompilerParams.__dataclass_fields__:
        _SC_PARAMS = dataclasses.replace(_SC_PARAMS, **{_field: _val})


def _t0_body(h_ref, w1_ref, a1_ref, z1ext_ref, sd_ref):
    z1 = jnp.dot(h_ref[...], w1_ref[...], preferred_element_type=_f32)
    s1 = jnp.dot(z1, a1_ref[...], preferred_element_type=_f32)
    z1ext_ref[:, :IN] = z1
    ones = jnp.ones((N, 1), _f32)
    zp = jnp.zeros((N, 13), _f32)
    z1ext_ref[:, IN:IN + L] = jnp.concatenate(
        [ones, s1[:, 0:1], s1[:, 1:2], zp], axis=1)
    z1ext_ref[:, IN + L:] = jnp.concatenate(
        [ones, zp, jnp.zeros((N, 2), _f32)], axis=1)
    sd_ref[...] = jnp.concatenate(
        [s1[:, 2:4], jnp.zeros((N, 14), _f32)], axis=1)


def _t1_body(accp_ref, w2_ref, a2_ref, z2ext_ref, sd_ref):
    acc = accp_ref[0] + accp_ref[1]
    d0 = jnp.maximum(acc[:, IN:IN + 1], 1e-16)
    d1 = jnp.maximum(acc[:, IN + L:IN + L + 1], 1e-16)
    h1a = acc[:, :HID] / d0
    h1b = acc[:, HID:IN] / d1
    h1 = jnp.concatenate([h1a, h1b], axis=1)
    h1 = jnp.where(h1 >= 0, h1, 0.01 * h1)
    z2 = jnp.dot(h1, w2_ref[...], preferred_element_type=_f32)
    s2 = jnp.dot(z2, a2_ref[...], preferred_element_type=_f32)
    z2ext_ref[:, :OUT] = z2
    ones = jnp.ones((N, 1), _f32)
    z2ext_ref[:, OUT:] = jnp.concatenate(
        [ones, s2[:, 0:1], jnp.zeros((N, 14), _f32)], axis=1)
    sd_ref[...] = jnp.concatenate(
        [s2[:, 1:2], jnp.zeros((N, 15), _f32)], axis=1)


def _t2_body(accp_ref, wpt_ref, wpb_ref, bp_ref, p_ref):
    acc = accp_ref[0] + accp_ref[1]
    den = jnp.maximum(acc[:, OUT:OUT + 1], 1e-16)
    h2 = acc[:, :OUT] / den
    ps = jnp.dot(h2, wpt_ref[...], preferred_element_type=_f32)
    pd = jnp.dot(h2, wpb_ref[...], preferred_element_type=_f32) + bp_ref[...]
    p_ref[...] = jnp.concatenate([ps, pd], axis=1)


def _splat_i32(x):
    return jnp.broadcast_to(jnp.asarray(x, _i32), (L,))


def _leaky(x):
    return jnp.where(x >= 0, x, 0.01 * x)


def _build_keep(keep_v, del_v, elo):
    ones = jnp.full((L,), 1.0, _f32)
    zeros = jnp.full((L,), 0.0, _f32)

    @pl.loop(0, EPT, step=L)
    def _(i):
        keep_v[pl.ds(i, L)] = ones

    @pl.loop(0, DELP, step=L)
    def _(i):
        d16 = del_v[pl.ds(i, L)]
        loc = d16 - elo
        m = (loc >= 0) & (loc < EPT)
        locc = jnp.clip(loc, 0, EPT - 1)
        plsc.store_scatter(keep_v, [locc], zeros, mask=m)


def _bcast_f32(ref, e):
    return plsc.load_gather(ref, [jnp.broadcast_to(e, (L,)).astype(_i32)])


def _make_keep_pass():
    mesh = plsc.VectorSubcoreMesh(core_axis_name="c", subcore_axis_name="s")

    @functools.partial(
        pl.kernel,
        out_type=jax.ShapeDtypeStruct((E,), _f32),
        mesh=mesh,
        scratch_types=[
            pltpu.VMEM((EPT,), _f32),
            pltpu.VMEM((DELP,), _i32),
        ],
        compiler_params=_SC_PARAMS,
    )
    def keep_pass(del_hbm, keep_hbm, keep_v, del_v):
        cid = lax.axis_index("c")
        sid = lax.axis_index("s")
        wid = sid * 2 + cid
        elo = wid * EPT
        pltpu.sync_copy(del_hbm, del_v)
        _build_keep(keep_v, del_v, elo)
        pltpu.sync_copy(keep_v, keep_hbm.at[pl.ds(elo, EPT)])

    return keep_pass


_keep_pass = _make_keep_pass()


def _make_edge_pass(width, heads):
    nslice = width // L
    aux = width - 2 * L if heads == 2 else width - L
    mesh = plsc.VectorSubcoreMesh(core_axis_name="c", subcore_axis_name="s")

    @functools.partial(
        pl.kernel,
        out_type=jax.ShapeDtypeStruct((2, N, width), _f32),
        mesh=mesh,
        scratch_types=[
            pltpu.VMEM((CHUNK,), _i32),
            pltpu.VMEM((CHUNK,), _i32),
            pltpu.VMEM((CHUNK,), _f32),
            pltpu.VMEM((CHUNK, width), _f32),
            pltpu.VMEM((CHUNK, L), _f32),
            pltpu.VMEM((CHUNK,), _i32),
            pltpu.VMEM((CHUNK,), _i32),
            pltpu.VMEM((CHUNK,), _f32),
            pltpu.VMEM((CHUNK, width), _f32),
            pltpu.VMEM((CHUNK, L), _f32),
            pltpu.VMEM((CHUNK,), _f32),
            pltpu.VMEM((CHUNK,), _f32),
            pltpu.VMEM_SHARED((N, width), _f32),
            pltpu.SemaphoreType.DMA,
            pltpu.SemaphoreType.DMA,
            pltpu.SemaphoreType.DMA,
            pltpu.SemaphoreType.DMA,
            pltpu.SemaphoreType.DMA,
            pltpu.SemaphoreType.DMA,
        ],
        compiler_params=_SC_PARAMS,
    )
    def edge_pass(zext_hbm, sd_hbm, src_hbm, dst_hbm, keep_hbm, accp_hbm,
                  src_a, dst_a, kp_a, rows_a, sd_a,
                  src_b, dst_b, kp_b, rows_b, sd_b,
                  e0_v, e1_v, acc_sh, g_a, h_a, c_a, g_b, h_b, c_b):
        cid = lax.axis_index("c")
        sid = lax.axis_index("s")
        wid = sid * 2 + cid
        elo = wid * EPT

        zeros = jnp.full((L,), 0.0, _f32)

        @pl.loop(0, CHUNK)
        def _(i):
            for s in range(nslice):
                rows_a[i, pl.ds(s * L, L)] = zeros

        @pl.when(sid < 10)
        def _():
            for k in range(12):
                pltpu.sync_copy(
                    rows_a, acc_sh.at[pl.ds(sid * 1000 + k * CHUNK, CHUNK)])
            pltpu.sync_copy(rows_a.at[pl.ds(0, 40)],
                            acc_sh.at[pl.ds(sid * 1000 + 960, 40)])
        plsc.subcore_barrier()

        cs0 = _splat_i32(aux + 1)
        cs1 = _splat_i32(aux + 2)
        cd0 = _splat_i32(0)
        cd1 = _splat_i32(1)
        iota = lax.iota(_i32, L)

        def small(c, src_x, dst_x, kp_x):
            pltpu.sync_copy(src_hbm.at[wid].at[c], src_x)
            pltpu.sync_copy(dst_hbm.at[wid].at[c], dst_x)
            pltpu.sync_copy(keep_hbm.at[pl.ds(elo + c * CHUNK, CHUNK)], kp_x)

        def fire(src_x, dst_x, rows_x, sd_x, g_x, h_x):
            pltpu.async_copy(zext_hbm.at[src_x], rows_x, g_x)
            pltpu.async_copy(sd_hbm.at[dst_x], sd_x, h_x)

        def wait_gather(src_x, dst_x, rows_x, sd_x, g_x, h_x):
            pltpu.make_async_copy(zext_hbm.at[src_x], rows_x, g_x).wait()
            pltpu.make_async_copy(sd_hbm.at[dst_x], sd_x, h_x).wait()

        def wait_scatter(dst_x, rows_x, c_x):
            pltpu.make_async_copy(rows_x, acc_sh.at[dst_x], c_x).wait()

        def proc(dst_x, kp_x, rows_x, sd_x, c_x):
            @pl.loop(0, CHUNK // L)
            def grp(g):
                idx16 = iota + g * L
                k16 = kp_x[pl.ds(g * L, L)]
                e0 = (plsc.load_gather(rows_x, [idx16, cs0])
                      + plsc.load_gather(sd_x, [idx16, cd0]))
                e0_v[pl.ds(g * L, L)] = jnp.exp(_leaky(e0)) * k16
                if heads == 2:
                    e1 = (plsc.load_gather(rows_x, [idx16, cs1])
                          + plsc.load_gather(sd_x, [idx16, cd1]))
                    e1_v[pl.ds(g * L, L)] = jnp.exp(_leaky(e1)) * k16

            @pl.loop(0, CHUNK, step=2)
            def srow(e):
                for d in range(2):
                    b0 = _bcast_f32(e0_v, e + d)
                    if heads == 2:
                        b1 = _bcast_f32(e1_v, e + d)
                        for s in range(nslice):
                            b = b0 if (s < 4 or s == nslice - 2) else b1
                            rows_x[e + d, pl.ds(s * L, L)] = (
                                rows_x[e + d, pl.ds(s * L, L)] * b)
                    else:
                        for s in range(nslice):
                            rows_x[e + d, pl.ds(s * L, L)] = (
                                rows_x[e + d, pl.ds(s * L, L)] * b0)

            pltpu.async_copy(rows_x, acc_sh.at[dst_x], c_x, add=True)

        small(0, src_a, dst_a, kp_a)
        fire(src_a, dst_a, rows_a, sd_a, g_a, h_a)

        @pl.loop(0, NCHUNK - 1, step=2)
        def pair(c):
            @pl.when(c >= 1)
            def _():
                wait_scatter(dst_b, rows_b, c_b)
            small(c + 1, src_b, dst_b, kp_b)
            fire(src_b, dst_b, rows_b, sd_b, g_b, h_b)
            wait_gather(src_a, dst_a, rows_a, sd_a, g_a, h_a)
            proc(dst_a, kp_a, rows_a, sd_a, c_a)
            wait_gather(src_b, dst_b, rows_b, sd_b, g_b, h_b)
            proc(dst_b, kp_b, rows_b, sd_b, c_b)
            wait_scatter(dst_a, rows_a, c_a)
            small(c + 2, src_a, dst_a, kp_a)
            fire(src_a, dst_a, rows_a, sd_a, g_a, h_a)

        wait_scatter(dst_b, rows_b, c_b)
        wait_gather(src_a, dst_a, rows_a, sd_a, g_a, h_a)
        proc(dst_a, kp_a, rows_a, sd_a, c_a)
        wait_scatter(dst_a, rows_a, c_a)

        plsc.subcore_barrier()

        @pl.when(sid < 10)
        def _():
            for k in range(12):
                r0 = sid * 1000 + k * CHUNK
                pltpu.sync_copy(acc_sh.at[pl.ds(r0, CHUNK)], rows_a)
                pltpu.sync_copy(rows_a, accp_hbm.at[cid].at[pl.ds(r0, CHUNK)])
            r0 = sid * 1000 + 960
            pltpu.sync_copy(acc_sh.at[pl.ds(r0, 40)], rows_a.at[pl.ds(0, 40)])
            pltpu.sync_copy(rows_a.at[pl.ds(0, 40)],
                            accp_hbm.at[cid].at[pl.ds(r0, 40)])

    return edge_pass


_edge_pass1 = _make_edge_pass(W1EXT, 2)
_edge_pass2 = _make_edge_pass(W2EXT, 1)


def _make_score_pass():
    mesh = plsc.VectorSubcoreMesh(core_axis_name="c", subcore_axis_name="s")

    @functools.partial(
        pl.kernel,
        out_type=jax.ShapeDtypeStruct((E, 2), _f32),
        mesh=mesh,
        scratch_types=[
            pltpu.VMEM((N, 4), _f32),
            pltpu.VMEM((CHUNK,), _i32),
            pltpu.VMEM((CHUNK,), _i32),
            pltpu.VMEM((CHUNK, 2), _f32),
        ],
        compiler_params=_SC_PARAMS,
    )
    def score_pass(p_hbm, src_hbm, dst_hbm, score_hbm, p_v, src_v, dst_v,
                   out_buf):
        cid = lax.axis_index("c")
        sid = lax.axis_index("s")
        wid = sid * 2 + cid
        elo = wid * EPT

        pltpu.sync_copy(p_hbm, p_v)

        c0 = _splat_i32(0)
        c1 = _splat_i32(1)
        c2 = _splat_i32(2)
        c3 = _splat_i32(3)
        iota = lax.iota(_i32, L)

        @pl.loop(0, NCHUNK)
        def chunk(c):
            pltpu.sync_copy(src_hbm.at[wid].at[c], src_v)
            pltpu.sync_copy(dst_hbm.at[wid].at[c], dst_v)

            @pl.loop(0, CHUNK // L)
            def grp(g):
                s16 = src_v[pl.ds(g * L, L)]
                d16 = dst_v[pl.ds(g * L, L)]
                v0 = (plsc.load_gather(p_v, [s16, c0])
                      + plsc.load_gather(p_v, [d16, c2]))
                v1 = (plsc.load_gather(p_v, [s16, c1])
                      + plsc.load_gather(p_v, [d16, c3]))
                rows = iota + g * L
                plsc.store_scatter(out_buf, [rows, c0], v0)
                plsc.store_scatter(out_buf, [rows, c1], v1)

            pltpu.sync_copy(out_buf, score_hbm.at[pl.ds(elo + c * CHUNK, CHUNK)])

    return score_pass


_score_pass = _make_score_pass()


def _t0(h, w1cat, a1m):
    return pl.pallas_call(
        _t0_body,
        out_shape=[
            jax.ShapeDtypeStruct((N, W1EXT), _f32),
            jax.ShapeDtypeStruct((N, L), _f32),
        ],
    )(h, w1cat, a1m)


def _t1(accp, w2, a2m):
    return pl.pallas_call(
        _t1_body,
        out_shape=[
            jax.ShapeDtypeStruct((N, W2EXT), _f32),
            jax.ShapeDtypeStruct((N, L), _f32),
        ],
    )(accp, w2, a2m)


def _t2(accp, wpt, wpb, bp2):
    return pl.pallas_call(
        _t2_body,
        out_shape=jax.ShapeDtypeStruct((N, 4), _f32),
    )(accp, wpt, wpb, bp2)


@jax.jit
def kernel(h, edge_index, delete_eids, W1, a1, W2, a2, Wp, bp):
    src_r = edge_index[0].astype(_i32).reshape(NTILES, NCHUNK, CHUNK)
    dst_r = edge_index[1].astype(_i32).reshape(NTILES, NCHUNK, CHUNK)
    del_pad = jnp.concatenate(
        [delete_eids.astype(_i32),
         jnp.broadcast_to(delete_eids[:1].astype(_i32), (DELP - DEL,))])

    w1cat = jnp.concatenate([W1[0], W1[1]], axis=1)
    a1m = jnp.zeros((IN, 4), _f32)
    a1m = a1m.at[:HID, 0].set(a1[0, :HID, 0])
    a1m = a1m.at[HID:, 1].set(a1[1, :HID, 0])
    a1m = a1m.at[:HID, 2].set(a1[0, HID:, 0])
    a1m = a1m.at[HID:, 3].set(a1[1, HID:, 0])
    a2m = jnp.stack([a2[:OUT, 0], a2[OUT:, 0]], axis=1)
    wpt = Wp[:OUT]
    wpb = Wp[OUT:]
    bp2 = bp.reshape(1, C)

    keepf = _keep_pass(del_pad)
    z1ext, s1 = _t0(h, w1cat, a1m)
    accp1 = _edge_pass1(z1ext, s1, src_r, dst_r, keepf)
    z2ext, s2 = _t1(accp1, W2, a2m)
    accp2 = _edge_pass2(z2ext, s2, src_r, dst_r, keepf)
    p = _t2(accp2, wpt, wpb, bp2)
    score = _score_pass(p, src_r, dst_r)
    return score


# speedup vs baseline: 18.6957x; 1.0087x over previous
import dataclasses
import functools

import jax
import jax.numpy as jnp
from jax import lax
from jax.experimental import pallas as pl
from jax.experimental.pallas import tpu as pltpu
from jax.experimental.pallas import tpu_sc as plsc

N = 10000
E = 320000
IN = 128
HID = 64
HEADS = 2
OUT = 64
C = 2
DEL = 1000

L = 16
NTILES = 32
EPT = E // NTILES
CHUNK = 80
NCHUNK = EPT // CHUNK
DELP = 1008

W1EXT = 160
W2EXT = 80

_f32 = jnp.float32
_i32 = jnp.int32

_SC_PARAMS = pltpu.CompilerParams()
for _field, _val in (("needs_layout_passes", False),
                     ("use_tc_tiling_on_sc", False)):
    if _field in pltpu.CompilerParams.__dataclass_fields__:
        _SC_PARAMS = dataclasses.replace(_SC_PARAMS, **{_field: _val})


def _t0_body(h_ref, w1_ref, a1_ref, z1ext_ref, sd_ref):
    z1 = jnp.dot(h_ref[...], w1_ref[...], preferred_element_type=_f32)
    s1 = jnp.dot(z1, a1_ref[...], preferred_element_type=_f32)
    z1ext_ref[:, :IN] = z1
    ones = jnp.ones((N, 1), _f32)
    zp = jnp.zeros((N, 13), _f32)
    z1ext_ref[:, IN:IN + L] = jnp.concatenate(
        [ones, s1[:, 0:1], s1[:, 1:2], zp], axis=1)
    z1ext_ref[:, IN + L:] = jnp.concatenate(
        [ones, zp, jnp.zeros((N, 2), _f32)], axis=1)
    sd_ref[...] = jnp.concatenate(
        [s1[:, 2:4], jnp.zeros((N, 14), _f32)], axis=1)


def _t1_body(accp_ref, w2_ref, a2_ref, z2ext_ref, sd_ref):
    acc = accp_ref[0] + accp_ref[1]
    d0 = jnp.maximum(acc[:, IN:IN + 1], 1e-16)
    d1 = jnp.maximum(acc[:, IN + L:IN + L + 1], 1e-16)
    h1a = acc[:, :HID] / d0
    h1b = acc[:, HID:IN] / d1
    h1 = jnp.concatenate([h1a, h1b], axis=1)
    h1 = jnp.where(h1 >= 0, h1, 0.01 * h1)
    z2 = jnp.dot(h1, w2_ref[...], preferred_element_type=_f32)
    s2 = jnp.dot(z2, a2_ref[...], preferred_element_type=_f32)
    z2ext_ref[:, :OUT] = z2
    ones = jnp.ones((N, 1), _f32)
    z2ext_ref[:, OUT:] = jnp.concatenate(
        [ones, s2[:, 0:1], jnp.zeros((N, 14), _f32)], axis=1)
    sd_ref[...] = jnp.concatenate(
        [s2[:, 1:2], jnp.zeros((N, 15), _f32)], axis=1)


def _t2_body(accp_ref, wpt_ref, wpb_ref, bp_ref, p_ref):
    acc = accp_ref[0] + accp_ref[1]
    den = jnp.maximum(acc[:, OUT:OUT + 1], 1e-16)
    h2 = acc[:, :OUT] / den
    ps = jnp.dot(h2, wpt_ref[...], preferred_element_type=_f32)
    pd = jnp.dot(h2, wpb_ref[...], preferred_element_type=_f32) + bp_ref[...]
    p_ref[...] = jnp.concatenate([ps, pd], axis=1)


def _splat_i32(x):
    return jnp.broadcast_to(jnp.asarray(x, _i32), (L,))


def _leaky(x):
    return jnp.where(x >= 0, x, 0.01 * x)


def _build_keep(keep_v, del_v, elo):
    ones = jnp.full((L,), 1.0, _f32)
    zeros = jnp.full((L,), 0.0, _f32)

    @pl.loop(0, EPT, step=L)
    def _(i):
        keep_v[pl.ds(i, L)] = ones

    @pl.loop(0, DELP, step=L)
    def _(i):
        d16 = del_v[pl.ds(i, L)]
        loc = d16 - elo
        m = (loc >= 0) & (loc < EPT)
        locc = jnp.clip(loc, 0, EPT - 1)
        plsc.store_scatter(keep_v, [locc], zeros, mask=m)


def _bcast_f32(ref, e):
    return plsc.load_gather(ref, [jnp.broadcast_to(e, (L,)).astype(_i32)])


def _make_keep_pass():
    mesh = plsc.VectorSubcoreMesh(core_axis_name="c", subcore_axis_name="s")

    @functools.partial(
        pl.kernel,
        out_type=jax.ShapeDtypeStruct((E,), _f32),
        mesh=mesh,
        scratch_types=[
            pltpu.VMEM((EPT,), _f32),
            pltpu.VMEM((DELP,), _i32),
        ],
        compiler_params=_SC_PARAMS,
    )
    def keep_pass(del_hbm, keep_hbm, keep_v, del_v):
        cid = lax.axis_index("c")
        sid = lax.axis_index("s")
        wid = sid * 2 + cid
        elo = wid * EPT
        pltpu.sync_copy(del_hbm, del_v)
        _build_keep(keep_v, del_v, elo)
        pltpu.sync_copy(keep_v, keep_hbm.at[pl.ds(elo, EPT)])

    return keep_pass


_keep_pass = _make_keep_pass()


def _make_edge_pass(width, heads):
    nslice = width // L
    aux = width - 2 * L if heads == 2 else width - L
    mesh = plsc.VectorSubcoreMesh(core_axis_name="c", subcore_axis_name="s")

    @functools.partial(
        pl.kernel,
        out_type=jax.ShapeDtypeStruct((2, N, width), _f32),
        mesh=mesh,
        scratch_types=[
            pltpu.VMEM((CHUNK,), _i32),
            pltpu.VMEM((CHUNK,), _i32),
            pltpu.VMEM((CHUNK,), _f32),
            pltpu.VMEM((CHUNK, width), _f32),
            pltpu.VMEM((CHUNK, L), _f32),
            pltpu.VMEM((CHUNK,), _i32),
            pltpu.VMEM((CHUNK,), _i32),
            pltpu.VMEM((CHUNK,), _f32),
            pltpu.VMEM((CHUNK, width), _f32),
            pltpu.VMEM((CHUNK, L), _f32),
            pltpu.VMEM((CHUNK,), _f32),
            pltpu.VMEM((CHUNK,), _f32),
            pltpu.VMEM_SHARED((N, width), _f32),
            pltpu.SemaphoreType.DMA,
            pltpu.SemaphoreType.DMA,
            pltpu.SemaphoreType.DMA,
            pltpu.SemaphoreType.DMA,
            pltpu.SemaphoreType.DMA,
            pltpu.SemaphoreType.DMA,
        ],
        compiler_params=_SC_PARAMS,
    )
    def edge_pass(zext_hbm, sd_hbm, src_hbm, dst_hbm, keep_hbm, accp_hbm,
                  src_a, dst_a, kp_a, rows_a, sd_a,
                  src_b, dst_b, kp_b, rows_b, sd_b,
                  e0_v, e1_v, acc_sh, g_a, h_a, c_a, g_b, h_b, c_b):
        cid = lax.axis_index("c")
        sid = lax.axis_index("s")
        wid = sid * 2 + cid
        elo = wid * EPT

        zeros = jnp.full((L,), 0.0, _f32)

        @pl.loop(0, CHUNK)
        def _(i):
            for s in range(nslice):
                rows_a[i, pl.ds(s * L, L)] = zeros

        @pl.when(sid < 10)
        def _():
            for k in range(12):
                pltpu.sync_copy(
                    rows_a, acc_sh.at[pl.ds(sid * 1000 + k * CHUNK, CHUNK)])
            pltpu.sync_copy(rows_a.at[pl.ds(0, 40)],
                            acc_sh.at[pl.ds(sid * 1000 + 960, 40)])
        plsc.subcore_barrier()

        cs0 = _splat_i32(aux + 1)
        cs1 = _splat_i32(aux + 2)
        cd0 = _splat_i32(0)
        cd1 = _splat_i32(1)
        iota = lax.iota(_i32, L)

        def small(c, src_x, dst_x, kp_x):
            pltpu.sync_copy(src_hbm.at[wid].at[c], src_x)
            pltpu.sync_copy(dst_hbm.at[wid].at[c], dst_x)
            pltpu.sync_copy(keep_hbm.at[pl.ds(elo + c * CHUNK, CHUNK)], kp_x)

        def fire(src_x, dst_x, rows_x, sd_x, g_x, h_x):
            pltpu.async_copy(zext_hbm.at[src_x], rows_x, g_x)
            pltpu.async_copy(sd_hbm.at[dst_x], sd_x, h_x)

        def wait_gather(src_x, dst_x, rows_x, sd_x, g_x, h_x):
            pltpu.make_async_copy(zext_hbm.at[src_x], rows_x, g_x).wait()
            pltpu.make_async_copy(sd_hbm.at[dst_x], sd_x, h_x).wait()

        def wait_scatter(dst_x, rows_x, c_x):
            pltpu.make_async_copy(rows_x, acc_sh.at[dst_x], c_x).wait()

        def proc(dst_x, kp_x, rows_x, sd_x, c_x):
            @pl.loop(0, CHUNK // L)
            def grp(g):
                idx16 = iota + g * L
                k16 = kp_x[pl.ds(g * L, L)]
                e0 = (plsc.load_gather(rows_x, [idx16, cs0])
                      + plsc.load_gather(sd_x, [idx16, cd0]))
                e0_v[pl.ds(g * L, L)] = jnp.exp(_leaky(e0)) * k16
                if heads == 2:
                    e1 = (plsc.load_gather(rows_x, [idx16, cs1])
                          + plsc.load_gather(sd_x, [idx16, cd1]))
                    e1_v[pl.ds(g * L, L)] = jnp.exp(_leaky(e1)) * k16

            @pl.loop(0, CHUNK, step=2)
            def srow(e):
                for d in range(2):
                    b0 = _bcast_f32(e0_v, e + d)
                    if heads == 2:
                        b1 = _bcast_f32(e1_v, e + d)
                        for s in range(nslice):
                            b = b0 if (s < 4 or s == nslice - 2) else b1
                            rows_x[e + d, pl.ds(s * L, L)] = (
                                rows_x[e + d, pl.ds(s * L, L)] * b)
                    else:
                        for s in range(nslice):
                            rows_x[e + d, pl.ds(s * L, L)] = (
                                rows_x[e + d, pl.ds(s * L, L)] * b0)

            pltpu.async_copy(rows_x, acc_sh.at[dst_x], c_x, add=True)

        small(0, src_a, dst_a, kp_a)
        fire(src_a, dst_a, rows_a, sd_a, g_a, h_a)

        @pl.loop(0, NCHUNK - 1, step=2)
        def pair(c):
            @pl.when(c >= 1)
            def _():
                wait_scatter(dst_b, rows_b, c_b)
            small(c + 1, src_b, dst_b, kp_b)
            fire(src_b, dst_b, rows_b, sd_b, g_b, h_b)
            wait_gather(src_a, dst_a, rows_a, sd_a, g_a, h_a)
            proc(dst_a, kp_a, rows_a, sd_a, c_a)
            wait_gather(src_b, dst_b, rows_b, sd_b, g_b, h_b)
            proc(dst_b, kp_b, rows_b, sd_b, c_b)
            wait_scatter(dst_a, rows_a, c_a)
            small(c + 2, src_a, dst_a, kp_a)
            fire(src_a, dst_a, rows_a, sd_a, g_a, h_a)

        wait_scatter(dst_b, rows_b, c_b)
        wait_gather(src_a, dst_a, rows_a, sd_a, g_a, h_a)
        proc(dst_a, kp_a, rows_a, sd_a, c_a)
        wait_scatter(dst_a, rows_a, c_a)

        plsc.subcore_barrier()

        @pl.when(sid < 10)
        def _():
            for k in range(12):
                r0 = sid * 1000 + k * CHUNK
                pltpu.sync_copy(acc_sh.at[pl.ds(r0, CHUNK)], rows_a)
                pltpu.sync_copy(rows_a, accp_hbm.at[cid].at[pl.ds(r0, CHUNK)])
            r0 = sid * 1000 + 960
            pltpu.sync_copy(acc_sh.at[pl.ds(r0, 40)], rows_a.at[pl.ds(0, 40)])
            pltpu.sync_copy(rows_a.at[pl.ds(0, 40)],
                            accp_hbm.at[cid].at[pl.ds(r0, 40)])

    return edge_pass


_edge_pass1 = _make_edge_pass(W1EXT, 2)
_edge_pass2 = _make_edge_pass(W2EXT, 1)


def _make_score_pass():
    mesh = plsc.VectorSubcoreMesh(core_axis_name="c", subcore_axis_name="s")

    @functools.partial(
        pl.kernel,
        out_type=jax.ShapeDtypeStruct((E, 2), _f32),
        mesh=mesh,
        scratch_types=[
            pltpu.VMEM((N, 4), _f32),
            pltpu.VMEM((CHUNK,), _i32),
            pltpu.VMEM((CHUNK,), _i32),
            pltpu.VMEM((CHUNK, 2), _f32),
            pltpu.VMEM((CHUNK,), _i32),
            pltpu.VMEM((CHUNK,), _i32),
            pltpu.VMEM((CHUNK, 2), _f32),
            pltpu.SemaphoreType.DMA,
            pltpu.SemaphoreType.DMA,
        ],
        compiler_params=_SC_PARAMS,
    )
    def score_pass(p_hbm, src_hbm, dst_hbm, score_hbm, p_v,
                   src_a, dst_a, out_a, src_b, dst_b, out_b, o_a, o_b):
        cid = lax.axis_index("c")
        sid = lax.axis_index("s")
        wid = sid * 2 + cid
        elo = wid * EPT

        pltpu.sync_copy(p_hbm, p_v)

        c0 = _splat_i32(0)
        c1 = _splat_i32(1)
        c2 = _splat_i32(2)
        c3 = _splat_i32(3)
        iota = lax.iota(_i32, L)

        def small(c, src_x, dst_x):
            pltpu.sync_copy(src_hbm.at[wid].at[c], src_x)
            pltpu.sync_copy(dst_hbm.at[wid].at[c], dst_x)

        def out_slice(c):
            return score_hbm.at[pl.ds(elo + c * CHUNK, CHUNK)]

        def procS(c, src_x, dst_x, out_x, o_x):
            @pl.loop(0, CHUNK // L)
            def grp(g):
                s16 = src_x[pl.ds(g * L, L)]
                d16 = dst_x[pl.ds(g * L, L)]
                v0 = (plsc.load_gather(p_v, [s16, c0])
                      + plsc.load_gather(p_v, [d16, c2]))
                v1 = (plsc.load_gather(p_v, [s16, c1])
                      + plsc.load_gather(p_v, [d16, c3]))
                rows = iota + g * L
                plsc.store_scatter(out_x, [rows, c0], v0)
                plsc.store_scatter(out_x, [rows, c1], v1)
            pltpu.async_copy(out_x, out_slice(c), o_x)

        small(0, src_a, dst_a)

        @pl.loop(0, NCHUNK - 1, step=2)
        def pair(c):
            small(c + 1, src_b, dst_b)
            @pl.when(c >= 2)
            def _():
                pltpu.make_async_copy(out_a, out_slice(c - 2), o_a).wait()
            procS(c, src_a, dst_a, out_a, o_a)
            small(c + 2, src_a, dst_a)
            @pl.when(c >= 1)
            def _():
                pltpu.make_async_copy(out_b, out_slice(c - 1), o_b).wait()
            procS(c + 1, src_b, dst_b, out_b, o_b)

        pltpu.make_async_copy(out_a, out_slice(NCHUNK - 3), o_a).wait()
        procS(NCHUNK - 1, src_a, dst_a, out_a, o_a)
        pltpu.make_async_copy(out_b, out_slice(NCHUNK - 2), o_b).wait()
        pltpu.make_async_copy(out_a, out_slice(NCHUNK - 1), o_a).wait()

    return score_pass


_score_pass = _make_score_pass()


def _t0(h, w1cat, a1m):
    return pl.pallas_call(
        _t0_body,
        out_shape=[
            jax.ShapeDtypeStruct((N, W1EXT), _f32),
            jax.ShapeDtypeStruct((N, L), _f32),
        ],
    )(h, w1cat, a1m)


def _t1(accp, w2, a2m):
    return pl.pallas_call(
        _t1_body,
        out_shape=[
            jax.ShapeDtypeStruct((N, W2EXT), _f32),
            jax.ShapeDtypeStruct((N, L), _f32),
        ],
    )(accp, w2, a2m)


def _t2(accp, wpt, wpb, bp2):
    return pl.pallas_call(
        _t2_body,
        out_shape=jax.ShapeDtypeStruct((N, 4), _f32),
    )(accp, wpt, wpb, bp2)


@jax.jit
def kernel(h, edge_index, delete_eids, W1, a1, W2, a2, Wp, bp):
    src_r = edge_index[0].astype(_i32).reshape(NTILES, NCHUNK, CHUNK)
    dst_r = edge_index[1].astype(_i32).reshape(NTILES, NCHUNK, CHUNK)
    del_pad = jnp.concatenate(
        [delete_eids.astype(_i32),
         jnp.broadcast_to(delete_eids[:1].astype(_i32), (DELP - DEL,))])

    w1cat = jnp.concatenate([W1[0], W1[1]], axis=1)
    a1m = jnp.zeros((IN, 4), _f32)
    a1m = a1m.at[:HID, 0].set(a1[0, :HID, 0])
    a1m = a1m.at[HID:, 1].set(a1[1, :HID, 0])
    a1m = a1m.at[:HID, 2].set(a1[0, HID:, 0])
    a1m = a1m.at[HID:, 3].set(a1[1, HID:, 0])
    a2m = jnp.stack([a2[:OUT, 0], a2[OUT:, 0]], axis=1)
    wpt = Wp[:OUT]
    wpb = Wp[OUT:]
    bp2 = bp.reshape(1, C)

    keepf = _keep_pass(del_pad)
    z1ext, s1 = _t0(h, w1cat, a1m)
    accp1 = _edge_pass1(z1ext, s1, src_r, dst_r, keepf)
    z2ext, s2 = _t1(accp1, W2, a2m)
    accp2 = _edge_pass2(z2ext, s2, src_r, dst_r, keepf)
    p = _t2(accp2, wpt, wpb, bp2)
    score = _score_pass(p, src_r, dst_r)
    return score


# speedup vs baseline: 18.9410x; 1.0131x over previous
import dataclasses
import functools

import jax
import jax.numpy as jnp
from jax import lax
from jax.experimental import pallas as pl
from jax.experimental.pallas import tpu as pltpu
from jax.experimental.pallas import tpu_sc as plsc

N = 10000
E = 320000
IN = 128
HID = 64
HEADS = 2
OUT = 64
C = 2
DEL = 1000

L = 16
NTILES = 32
EPT = E // NTILES
CHUNK = 80
NCHUNK = EPT // CHUNK
DELP = 1008

W1EXT = 144
W2EXT = 80

_f32 = jnp.float32
_i32 = jnp.int32

_SC_PARAMS = pltpu.CompilerParams()
for _field, _val in (("needs_layout_passes", False),
                     ("use_tc_tiling_on_sc", False)):
    if _field in pltpu.CompilerParams.__dataclass_fields__:
        _SC_PARAMS = dataclasses.replace(_SC_PARAMS, **{_field: _val})


def _t0_body(h_ref, w1_ref, a1_ref, z1ext_ref, sd_ref):
    z1 = jnp.dot(h_ref[...], w1_ref[...], preferred_element_type=_f32)
    s1 = jnp.dot(z1, a1_ref[...], preferred_element_type=_f32)
    z1ext_ref[:, :IN] = z1
    ones = jnp.ones((N, 1), _f32)
    zp = jnp.zeros((N, 12), _f32)
    z1ext_ref[:, IN:] = jnp.concatenate(
        [ones, ones, s1[:, 0:1], s1[:, 1:2], zp], axis=1)
    sd_ref[...] = jnp.concatenate(
        [s1[:, 2:4], jnp.zeros((N, 14), _f32)], axis=1)


def _t1_body(accp_ref, w2_ref, a2_ref, z2ext_ref, sd_ref):
    acc = accp_ref[0] + accp_ref[1]
    d0 = jnp.maximum(acc[:, IN:IN + 1], 1e-16)
    d1 = jnp.maximum(acc[:, IN + 1:IN + 2], 1e-16)
    h1a = acc[:, :HID] / d0
    h1b = acc[:, HID:IN] / d1
    h1 = jnp.concatenate([h1a, h1b], axis=1)
    h1 = jnp.where(h1 >= 0, h1, 0.01 * h1)
    z2 = jnp.dot(h1, w2_ref[...], preferred_element_type=_f32)
    s2 = jnp.dot(z2, a2_ref[...], preferred_element_type=_f32)
    z2ext_ref[:, :OUT] = z2
    ones = jnp.ones((N, 1), _f32)
    z2ext_ref[:, OUT:] = jnp.concatenate(
        [ones, s2[:, 0:1], jnp.zeros((N, 14), _f32)], axis=1)
    sd_ref[...] = jnp.concatenate(
        [s2[:, 1:2], jnp.zeros((N, 15), _f32)], axis=1)


def _t2_body(accp_ref, wpt_ref, wpb_ref, bp_ref, p_ref):
    acc = accp_ref[0] + accp_ref[1]
    den = jnp.maximum(acc[:, OUT:OUT + 1], 1e-16)
    h2 = acc[:, :OUT] / den
    ps = jnp.dot(h2, wpt_ref[...], preferred_element_type=_f32)
    pd = jnp.dot(h2, wpb_ref[...], preferred_element_type=_f32) + bp_ref[...]
    p_ref[...] = jnp.concatenate([ps, pd], axis=1)


def _splat_i32(x):
    return jnp.broadcast_to(jnp.asarray(x, _i32), (L,))


def _leaky(x):
    return jnp.where(x >= 0, x, 0.01 * x)


def _build_keep(keep_v, del_v, elo):
    ones = jnp.full((L,), 1.0, _f32)
    zeros = jnp.full((L,), 0.0, _f32)

    @pl.loop(0, EPT, step=L)
    def _(i):
        keep_v[pl.ds(i, L)] = ones

    @pl.loop(0, DELP, step=L)
    def _(i):
        d16 = del_v[pl.ds(i, L)]
        loc = d16 - elo
        m = (loc >= 0) & (loc < EPT)
        locc = jnp.clip(loc, 0, EPT - 1)
        plsc.store_scatter(keep_v, [locc], zeros, mask=m)


def _bcast_f32(ref, e):
    return plsc.load_gather(ref, [jnp.broadcast_to(e, (L,)).astype(_i32)])


def _make_keep_pass():
    mesh = plsc.VectorSubcoreMesh(core_axis_name="c", subcore_axis_name="s")

    @functools.partial(
        pl.kernel,
        out_type=jax.ShapeDtypeStruct((E,), _f32),
        mesh=mesh,
        scratch_types=[
            pltpu.VMEM((EPT,), _f32),
            pltpu.VMEM((DELP,), _i32),
        ],
        compiler_params=_SC_PARAMS,
    )
    def keep_pass(del_hbm, keep_hbm, keep_v, del_v):
        cid = lax.axis_index("c")
        sid = lax.axis_index("s")
        wid = sid * 2 + cid
        elo = wid * EPT
        pltpu.sync_copy(del_hbm, del_v)
        _build_keep(keep_v, del_v, elo)
        pltpu.sync_copy(keep_v, keep_hbm.at[pl.ds(elo, EPT)])

    return keep_pass


_keep_pass = _make_keep_pass()


def _make_edge_pass(width, heads):
    nslice = width // L
    aux = width - L
    mesh = plsc.VectorSubcoreMesh(core_axis_name="c", subcore_axis_name="s")

    @functools.partial(
        pl.kernel,
        out_type=jax.ShapeDtypeStruct((2, N, width), _f32),
        mesh=mesh,
        scratch_types=[
            pltpu.VMEM((CHUNK,), _i32),
            pltpu.VMEM((CHUNK,), _i32),
            pltpu.VMEM((CHUNK,), _f32),
            pltpu.VMEM((CHUNK, width), _f32),
            pltpu.VMEM((CHUNK, L), _f32),
            pltpu.VMEM((CHUNK,), _i32),
            pltpu.VMEM((CHUNK,), _i32),
            pltpu.VMEM((CHUNK,), _f32),
            pltpu.VMEM((CHUNK, width), _f32),
            pltpu.VMEM((CHUNK, L), _f32),
            pltpu.VMEM((CHUNK,), _f32),
            pltpu.VMEM((CHUNK,), _f32),
            pltpu.VMEM_SHARED((N, width), _f32),
            pltpu.SemaphoreType.DMA,
            pltpu.SemaphoreType.DMA,
            pltpu.SemaphoreType.DMA,
            pltpu.SemaphoreType.DMA,
            pltpu.SemaphoreType.DMA,
            pltpu.SemaphoreType.DMA,
        ],
        compiler_params=_SC_PARAMS,
    )
    def edge_pass(zext_hbm, sd_hbm, src_hbm, dst_hbm, keep_hbm, accp_hbm,
                  src_a, dst_a, kp_a, rows_a, sd_a,
                  src_b, dst_b, kp_b, rows_b, sd_b,
                  e0_v, e1_v, acc_sh, g_a, h_a, c_a, g_b, h_b, c_b):
        cid = lax.axis_index("c")
        sid = lax.axis_index("s")
        wid = sid * 2 + cid
        elo = wid * EPT

        zeros = jnp.full((L,), 0.0, _f32)

        @pl.loop(0, CHUNK)
        def _(i):
            for s in range(nslice):
                rows_a[i, pl.ds(s * L, L)] = zeros

        @pl.when(sid < 10)
        def _():
            for k in range(12):
                pltpu.sync_copy(
                    rows_a, acc_sh.at[pl.ds(sid * 1000 + k * CHUNK, CHUNK)])
            pltpu.sync_copy(rows_a.at[pl.ds(0, 40)],
                            acc_sh.at[pl.ds(sid * 1000 + 960, 40)])
        plsc.subcore_barrier()

        cs0 = _splat_i32(aux + (2 if heads == 2 else 1))
        cs1 = _splat_i32(aux + 3)
        cd0 = _splat_i32(0)
        cd1 = _splat_i32(1)
        iota = lax.iota(_i32, L)

        def small(c, src_x, dst_x, kp_x):
            pltpu.sync_copy(src_hbm.at[wid].at[c], src_x)
            pltpu.sync_copy(dst_hbm.at[wid].at[c], dst_x)
            pltpu.sync_copy(keep_hbm.at[pl.ds(elo + c * CHUNK, CHUNK)], kp_x)

        def fire(src_x, dst_x, rows_x, sd_x, g_x, h_x):
            pltpu.async_copy(zext_hbm.at[src_x], rows_x, g_x)
            pltpu.async_copy(sd_hbm.at[dst_x], sd_x, h_x)

        def wait_gather(src_x, dst_x, rows_x, sd_x, g_x, h_x):
            pltpu.make_async_copy(zext_hbm.at[src_x], rows_x, g_x).wait()
            pltpu.make_async_copy(sd_hbm.at[dst_x], sd_x, h_x).wait()

        def wait_scatter(dst_x, rows_x, c_x):
            pltpu.make_async_copy(rows_x, acc_sh.at[dst_x], c_x).wait()

        def proc(dst_x, kp_x, rows_x, sd_x, c_x):
            @pl.loop(0, CHUNK // L)
            def grp(g):
                idx16 = iota + g * L
                k16 = kp_x[pl.ds(g * L, L)]
                e0 = (plsc.load_gather(rows_x, [idx16, cs0])
                      + plsc.load_gather(sd_x, [idx16, cd0]))
                e0_v[pl.ds(g * L, L)] = jnp.exp(_leaky(e0)) * k16
                if heads == 2:
                    e1 = (plsc.load_gather(rows_x, [idx16, cs1])
                          + plsc.load_gather(sd_x, [idx16, cd1]))
                    e1_v[pl.ds(g * L, L)] = jnp.exp(_leaky(e1)) * k16

            @pl.loop(0, CHUNK, step=2)
            def srow(e):
                for d in range(2):
                    b0 = _bcast_f32(e0_v, e + d)
                    if heads == 2:
                        b1 = _bcast_f32(e1_v, e + d)
                        maux = jnp.where(iota == 0, b0, b1)
                        for s in range(nslice):
                            b = (maux if s == nslice - 1
                                 else (b0 if s < 4 else b1))
                            rows_x[e + d, pl.ds(s * L, L)] = (
                                rows_x[e + d, pl.ds(s * L, L)] * b)
                    else:
                        for s in range(nslice):
                            rows_x[e + d, pl.ds(s * L, L)] = (
                                rows_x[e + d, pl.ds(s * L, L)] * b0)

            pltpu.async_copy(rows_x, acc_sh.at[dst_x], c_x, add=True)

        small(0, src_a, dst_a, kp_a)
        fire(src_a, dst_a, rows_a, sd_a, g_a, h_a)

        @pl.loop(0, NCHUNK - 1, step=2)
        def pair(c):
            @pl.when(c >= 1)
            def _():
                wait_scatter(dst_b, rows_b, c_b)
            small(c + 1, src_b, dst_b, kp_b)
            fire(src_b, dst_b, rows_b, sd_b, g_b, h_b)
            wait_gather(src_a, dst_a, rows_a, sd_a, g_a, h_a)
            proc(dst_a, kp_a, rows_a, sd_a, c_a)
            wait_gather(src_b, dst_b, rows_b, sd_b, g_b, h_b)
            proc(dst_b, kp_b, rows_b, sd_b, c_b)
            wait_scatter(dst_a, rows_a, c_a)
            small(c + 2, src_a, dst_a, kp_a)
            fire(src_a, dst_a, rows_a, sd_a, g_a, h_a)

        wait_scatter(dst_b, rows_b, c_b)
        wait_gather(src_a, dst_a, rows_a, sd_a, g_a, h_a)
        proc(dst_a, kp_a, rows_a, sd_a, c_a)
        wait_scatter(dst_a, rows_a, c_a)

        plsc.subcore_barrier()

        @pl.when(sid < 10)
        def _():
            for k in range(12):
                r0 = sid * 1000 + k * CHUNK
                pltpu.sync_copy(acc_sh.at[pl.ds(r0, CHUNK)], rows_a)
                pltpu.sync_copy(rows_a, accp_hbm.at[cid].at[pl.ds(r0, CHUNK)])
            r0 = sid * 1000 + 960
            pltpu.sync_copy(acc_sh.at[pl.ds(r0, 40)], rows_a.at[pl.ds(0, 40)])
            pltpu.sync_copy(rows_a.at[pl.ds(0, 40)],
                            accp_hbm.at[cid].at[pl.ds(r0, 40)])

    return edge_pass


_edge_pass1 = _make_edge_pass(W1EXT, 2)
_edge_pass2 = _make_edge_pass(W2EXT, 1)


def _make_score_pass():
    mesh = plsc.VectorSubcoreMesh(core_axis_name="c", subcore_axis_name="s")

    @functools.partial(
        pl.kernel,
        out_type=jax.ShapeDtypeStruct((E, 2), _f32),
        mesh=mesh,
        scratch_types=[
            pltpu.VMEM((N, 4), _f32),
            pltpu.VMEM((CHUNK,), _i32),
            pltpu.VMEM((CHUNK,), _i32),
            pltpu.VMEM((CHUNK, 2), _f32),
            pltpu.VMEM((CHUNK,), _i32),
            pltpu.VMEM((CHUNK,), _i32),
            pltpu.VMEM((CHUNK, 2), _f32),
            pltpu.SemaphoreType.DMA,
            pltpu.SemaphoreType.DMA,
        ],
        compiler_params=_SC_PARAMS,
    )
    def score_pass(p_hbm, src_hbm, dst_hbm, score_hbm, p_v,
                   src_a, dst_a, out_a, src_b, dst_b, out_b, o_a, o_b):
        cid = lax.axis_index("c")
        sid = lax.axis_index("s")
        wid = sid * 2 + cid
        elo = wid * EPT

        pltpu.sync_copy(p_hbm, p_v)

        c0 = _splat_i32(0)
        c1 = _splat_i32(1)
        c2 = _splat_i32(2)
        c3 = _splat_i32(3)
        iota = lax.iota(_i32, L)

        def small(c, src_x, dst_x):
            pltpu.sync_copy(src_hbm.at[wid].at[c], src_x)
            pltpu.sync_copy(dst_hbm.at[wid].at[c], dst_x)

        def out_slice(c):
            return score_hbm.at[pl.ds(elo + c * CHUNK, CHUNK)]

        def procS(c, src_x, dst_x, out_x, o_x):
            @pl.loop(0, CHUNK // L)
            def grp(g):
                s16 = src_x[pl.ds(g * L, L)]
                d16 = dst_x[pl.ds(g * L, L)]
                v0 = (plsc.load_gather(p_v, [s16, c0])
                      + plsc.load_gather(p_v, [d16, c2]))
                v1 = (plsc.load_gather(p_v, [s16, c1])
                      + plsc.load_gather(p_v, [d16, c3]))
                rows = iota + g * L
                plsc.store_scatter(out_x, [rows, c0], v0)
                plsc.store_scatter(out_x, [rows, c1], v1)
            pltpu.async_copy(out_x, out_slice(c), o_x)

        small(0, src_a, dst_a)

        @pl.loop(0, NCHUNK - 1, step=2)
        def pair(c):
            small(c + 1, src_b, dst_b)
            @pl.when(c >= 2)
            def _():
                pltpu.make_async_copy(out_a, out_slice(c - 2), o_a).wait()
            procS(c, src_a, dst_a, out_a, o_a)
            small(c + 2, src_a, dst_a)
            @pl.when(c >= 1)
            def _():
                pltpu.make_async_copy(out_b, out_slice(c - 1), o_b).wait()
            procS(c + 1, src_b, dst_b, out_b, o_b)

        pltpu.make_async_copy(out_a, out_slice(NCHUNK - 3), o_a).wait()
        procS(NCHUNK - 1, src_a, dst_a, out_a, o_a)
        pltpu.make_async_copy(out_b, out_slice(NCHUNK - 2), o_b).wait()
        pltpu.make_async_copy(out_a, out_slice(NCHUNK - 1), o_a).wait()

    return score_pass


_score_pass = _make_score_pass()


def _t0(h, w1cat, a1m):
    return pl.pallas_call(
        _t0_body,
        out_shape=[
            jax.ShapeDtypeStruct((N, W1EXT), _f32),
            jax.ShapeDtypeStruct((N, L), _f32),
        ],
    )(h, w1cat, a1m)


def _t1(accp, w2, a2m):
    return pl.pallas_call(
        _t1_body,
        out_shape=[
            jax.ShapeDtypeStruct((N, W2EXT), _f32),
            jax.ShapeDtypeStruct((N, L), _f32),
        ],
    )(accp, w2, a2m)


def _t2(accp, wpt, wpb, bp2):
    return pl.pallas_call(
        _t2_body,
        out_shape=jax.ShapeDtypeStruct((N, 4), _f32),
    )(accp, wpt, wpb, bp2)


@jax.jit
def kernel(h, edge_index, delete_eids, W1, a1, W2, a2, Wp, bp):
    src_r = edge_index[0].astype(_i32).reshape(NTILES, NCHUNK, CHUNK)
    dst_r = edge_index[1].astype(_i32).reshape(NTILES, NCHUNK, CHUNK)
    del_pad = jnp.concatenate(
        [delete_eids.astype(_i32),
         jnp.broadcast_to(delete_eids[:1].astype(_i32), (DELP - DEL,))])

    w1cat = jnp.concatenate([W1[0], W1[1]], axis=1)
    a1m = jnp.zeros((IN, 4), _f32)
    a1m = a1m.at[:HID, 0].set(a1[0, :HID, 0])
    a1m = a1m.at[HID:, 1].set(a1[1, :HID, 0])
    a1m = a1m.at[:HID, 2].set(a1[0, HID:, 0])
    a1m = a1m.at[HID:, 3].set(a1[1, HID:, 0])
    a2m = jnp.stack([a2[:OUT, 0], a2[OUT:, 0]], axis=1)
    wpt = Wp[:OUT]
    wpb = Wp[OUT:]
    bp2 = bp.reshape(1, C)

    keepf = _keep_pass(del_pad)
    z1ext, s1 = _t0(h, w1cat, a1m)
    accp1 = _edge_pass1(z1ext, s1, src_r, dst_r, keepf)
    z2ext, s2 = _t1(accp1, W2, a2m)
    accp2 = _edge_pass2(z2ext, s2, src_r, dst_r, keepf)
    p = _t2(accp2, wpt, wpb, bp2)
    score = _score_pass(p, src_r, dst_r)
    return score
